# baseline (speedup 1.0000x reference)
import jax
import jax.numpy as jnp
from jax import lax
from jax.experimental import pallas as pl
from jax.experimental.pallas import tpu as pltpu
from jax.experimental.pallas import tpu_sc as plsc

VOCAB = 76
SEQ = 2048
DMODEL = 512
BATCH = 64

NCORES = 2
NSUB = 16
NW = NCORES * NSUB
TPS = SEQ // NW
HB = BATCH // 2
NLANE = 16


def _pe_body(o_ref):
    r = lax.broadcasted_iota(jnp.int32, (SEQ, DMODEL), 0).astype(jnp.float32)
    c = lax.broadcasted_iota(jnp.int32, (SEQ, DMODEL), 1)
    even = (c - lax.rem(c, 2)).astype(jnp.float32)
    denom = jnp.exp(even * (jnp.log(10000.0) / DMODEL))
    theta = r / denom
    o_ref[...] = jnp.where(lax.rem(c, 2) == 0, jnp.sin(theta), jnp.cos(theta))


_pe_table = pl.pallas_call(
    _pe_body,
    out_shape=jax.ShapeDtypeStruct((SEQ, DMODEL), jnp.float32),
)


def _sc_body(idxT_hbm, table_hbm, pe_hbm, out_hbm,
             table_v, pe_v, idx_v, ob0, ob1, sem0, sem1):
    cid = lax.axis_index("c")
    sid = lax.axis_index("s")
    wid = sid * NCORES + cid
    pltpu.sync_copy(table_hbm, table_v)
    pltpu.sync_copy(pe_hbm.at[wid], pe_v)
    pltpu.sync_copy(idxT_hbm.at[wid], idx_v)

    bufs = ((ob0, sem0), (ob1, sem1))

    UJ = 8

    def compute(si, h, ob):
        def g_body(g, carry):
            tokvec = idx_v[si, pl.ds(h * HB + g * NLANE, NLANE)]

            def j_body(j0, carry2):
                for l in range(NLANE):
                    tok = tokvec[l]
                    row = g * NLANE + l
                    for ju in range(UJ):
                        sl = pl.ds((j0 * UJ + ju) * NLANE, NLANE)
                        ob[row, sl] = table_v[tok, sl] + pe_v[si, sl]
                return carry2

            lax.fori_loop(0, DMODEL // NLANE // UJ, j_body, 0)
            return carry

        lax.fori_loop(0, HB // NLANE, g_body, 0)

    def out_dst(si, h):
        return out_hbm.at[pl.ds(h * HB, HB), wid, si]

    def s_body(si, carry):
        for h, (ob, sem) in enumerate(bufs):
            @pl.when(si > 0)
            def _wait():
                pltpu.make_async_copy(ob, out_dst(si, h), sem).wait()

            compute(si, h, ob)
            pltpu.async_copy(ob, out_dst(si, h), sem)
        return carry

    lax.fori_loop(0, TPS, s_body, 0)

    for h, (ob, sem) in enumerate(bufs):
        pltpu.make_async_copy(ob, out_dst(TPS - 1, h), sem).wait()


_sc_embed = pl.kernel(
    _sc_body,
    out_type=jax.ShapeDtypeStruct((BATCH, NW, TPS, DMODEL), jnp.float32),
    mesh=plsc.VectorSubcoreMesh(core_axis_name="c", subcore_axis_name="s",
                                num_cores=NCORES, num_subcores=NSUB),
    scratch_types=[
        pltpu.VMEM((VOCAB, DMODEL), jnp.float32),
        pltpu.VMEM((TPS, DMODEL), jnp.float32),
        pltpu.VMEM((TPS, BATCH), jnp.int32),
        pltpu.VMEM((HB, DMODEL), jnp.float32),
        pltpu.VMEM((HB, DMODEL), jnp.float32),
        pltpu.SemaphoreType.DMA,
        pltpu.SemaphoreType.DMA,
    ],
)


def kernel(x, embedding):
    idxT = x.astype(jnp.int32).T.reshape(NW, TPS, BATCH)
    pe = _pe_table().reshape(NW, TPS, DMODEL)
    out = _sc_embed(idxT, embedding, pe)
    return out.reshape(BATCH, SEQ, DMODEL)
